# baseline (speedup 1.0000x reference)
import jax
import jax.numpy as jnp
from jax import lax
from jax.experimental import pallas as pl
from jax.experimental.pallas import tpu as pltpu
from jax.experimental.pallas import tpu_sc as plsc

B, F, T, K = 4096, 64, 128, 64
NC, NS, L = 2, 16, 16
NW = NC * NS
BPW = B // NW
ROW = F * T
NCHUNK = 4
KC = K // NCHUNK
NB = BPW // L


def _sc_body(x_ref, off_ref, out_ref, off_v, idx_v, val_v, out_v, osem, sems):
    wid = lax.axis_index("s") * NC + lax.axis_index("c")
    base = wid * BPW

    ocp = pltpu.async_copy(off_ref, off_v.at[pl.ds(0, K)], osem)
    lanes = lax.iota(jnp.int32, L)
    ocp.wait()

    copies = []
    for c in range(NCHUNK):
        def build(k, carry):
            off_k = off_v[pl.ds(k, L)][0]

            def row(b8, carry2):
                idx_v[pl.ds(k * BPW + b8 * L, L)] = (
                    (base + b8 * L + lanes) * ROW + off_k)
                return carry2

            return lax.fori_loop(0, NB, row, carry)

        lax.fori_loop(c * KC, (c + 1) * KC, build, 0)
        csl = pl.ds(c * KC * BPW, KC * BPW)
        cp = pltpu.async_copy(x_ref.at[idx_v.at[csl]], val_v.at[csl], sems[c])
        copies.append(cp)

    def zero(b8, carry):
        out_v[pl.ds(b8 * L, L)] = jnp.zeros((L,), jnp.float32)
        return carry

    lax.fori_loop(0, NB, zero, 0)

    for c in range(NCHUNK):
        copies[c].wait()

        def red(b8, carry):
            sl = pl.ds(b8 * L, L)

            def redk(k, acc):
                v = val_v[pl.ds(k * BPW + b8 * L, L)]
                return acc + v * v

            out_v[sl] = lax.fori_loop(c * KC, (c + 1) * KC, redk, out_v[sl])
            return carry

        lax.fori_loop(0, NB, red, 0)

    pltpu.sync_copy(out_v, out_ref.at[pl.ds(base, BPW)])


@jax.jit
def kernel(x, f_idx, t_idx):
    off = f_idx * T + t_idx
    kern = pl.kernel(
        _sc_body,
        out_type=jax.ShapeDtypeStruct((B,), jnp.float32),
        mesh=plsc.VectorSubcoreMesh(core_axis_name="c", subcore_axis_name="s"),
        scratch_types=[
            pltpu.VMEM((K + L,), jnp.int32),
            pltpu.VMEM((K * BPW,), jnp.int32),
            pltpu.VMEM((K * BPW,), jnp.float32),
            pltpu.VMEM((BPW,), jnp.float32),
            pltpu.SemaphoreType.DMA,
            [pltpu.SemaphoreType.DMA] * NCHUNK,
        ],
    )
    return kern(x.reshape(-1), off)


# speedup vs baseline: 1.0491x; 1.0491x over previous
import jax
import jax.numpy as jnp
from jax import lax
from jax.experimental import pallas as pl
from jax.experimental.pallas import tpu as pltpu
from jax.experimental.pallas import tpu_sc as plsc

B, F, T, K = 4096, 64, 128, 64
NC, NS, L = 2, 16, 16
NW = NC * NS
BPW = B // NW
ROW = F * T
KH = K // 2
BH = BPW // 2
NB4 = BH // L
CSZ = KH * BH


def _sc_body(x_ref, off_ref, out_ref, off_v, idx_v, val_v, out_v, osem,
             sems, wsems):
    wid = lax.axis_index("s") * NC + lax.axis_index("c")
    base = wid * BPW

    ocp = pltpu.async_copy(off_ref, off_v.at[pl.ds(0, K)], osem)
    lanes = lax.iota(jnp.int32, L)
    row_base = [(base + g * L + lanes) * ROW for g in range(BPW // L)]
    ocp.wait()

    copies = []
    for hb in range(2):
        for hk in range(2):
            c = hb * 2 + hk

            def build(kl, carry):
                off_k = off_v[pl.ds(hk * KH + kl, L)][0]
                for b4 in range(NB4):
                    idx_v[pl.ds(c * CSZ + kl * BH + b4 * L, L)] = (
                        row_base[hb * NB4 + b4] + off_k)
                return carry

            lax.fori_loop(0, KH, build, 0)
            csl = pl.ds(c * CSZ, CSZ)
            copies.append(pltpu.async_copy(
                x_ref.at[idx_v.at[csl]], val_v.at[csl], sems[c]))

    wcps = []
    for hb in range(2):
        accs = [jnp.zeros((L,), jnp.float32) for _ in range(NB4)]
        for hk in range(2):
            c = hb * 2 + hk
            copies[c].wait()

            def red(kl, accs):
                out = []
                for b4 in range(NB4):
                    v = val_v[pl.ds(c * CSZ + kl * BH + b4 * L, L)]
                    out.append(accs[b4] + v * v)
                return tuple(out)

            accs = lax.fori_loop(0, KH, red, tuple(accs))

        for b4 in range(NB4):
            out_v[pl.ds(hb * BH + b4 * L, L)] = accs[b4]
        wcps.append(pltpu.async_copy(
            out_v.at[pl.ds(hb * BH, BH)],
            out_ref.at[pl.ds(base + hb * BH, BH)], wsems[hb]))

    for wcp in wcps:
        wcp.wait()


@jax.jit
def kernel(x, f_idx, t_idx):
    off = f_idx * T + t_idx
    kern = pl.kernel(
        _sc_body,
        out_type=jax.ShapeDtypeStruct((B,), jnp.float32),
        mesh=plsc.VectorSubcoreMesh(core_axis_name="c", subcore_axis_name="s"),
        scratch_types=[
            pltpu.VMEM((K + L,), jnp.int32),
            pltpu.VMEM((K * BPW,), jnp.int32),
            pltpu.VMEM((K * BPW,), jnp.float32),
            pltpu.VMEM((BPW,), jnp.float32),
            pltpu.SemaphoreType.DMA,
            [pltpu.SemaphoreType.DMA] * 4,
            [pltpu.SemaphoreType.DMA] * 2,
        ],
    )
    return kern(x.reshape(-1), off)
